# baseline (speedup 1.0000x reference)
import functools

import jax
import jax.numpy as jnp
from jax import lax
from jax.experimental import pallas as pl
from jax.experimental.pallas import tpu as pltpu
from jax.experimental.pallas import tpu_sc as plsc

_D = 32
_NW = 32
_CHUNK = 1600
_NBUF = 2


def _make_gather(B):
    b_per_w = B // _NW
    n_chunk = b_per_w // _CHUNK
    mesh = plsc.VectorSubcoreMesh(core_axis_name="c", subcore_axis_name="s")

    @functools.partial(
        pl.kernel,
        mesh=mesh,
        out_type=jax.ShapeDtypeStruct((B, _D), jnp.float32),
        scratch_types=[
            pltpu.VMEM((2 * _NBUF, _CHUNK), jnp.int32),
            pltpu.VMEM((_NBUF, _CHUNK, _D), jnp.float32),
            pltpu.SemaphoreType.DMA((2 * _NBUF,)),
            pltpu.SemaphoreType.DMA((_NBUF,)),
            pltpu.SemaphoreType.DMA((_NBUF,)),
        ],
        compiler_params=pltpu.CompilerParams(use_tc_tiling_on_sc=False),
    )
    def gather_kernel(idx_hbm, table_hbm, out_hbm, idx_v, rows_v,
                      sem_i, sem_g, sem_w):
        wid = lax.axis_index("s") * 2 + lax.axis_index("c")
        base = wid * b_per_w

        def idx_cp(g):
            bi = g % (2 * _NBUF)
            return pltpu.make_async_copy(
                idx_hbm.at[pl.ds(base + g * _CHUNK, _CHUNK)],
                idx_v.at[bi], sem_i.at[bi])

        def gath(g):
            b = g % _NBUF
            return pltpu.make_async_copy(
                table_hbm.at[idx_v.at[g % (2 * _NBUF)]],
                rows_v.at[b], sem_g.at[b])

        def wb(g):
            b = g % _NBUF
            return pltpu.make_async_copy(
                rows_v.at[b],
                out_hbm.at[pl.ds(base + g * _CHUNK, _CHUNK)], sem_w.at[b])

        for g in range(min(2 * _NBUF, n_chunk)):
            idx_cp(g).start()
        for g in range(n_chunk):
            if g >= _NBUF:
                wb(g - _NBUF).wait()
            idx_cp(g).wait()
            gath(g).start()
            gath(g).wait()
            if g + 2 * _NBUF < n_chunk:
                idx_cp(g + 2 * _NBUF).start()
            wb(g).start()
        for g in range(max(0, n_chunk - _NBUF), n_chunk):
            wb(g).wait()

    return gather_kernel


def kernel(x, embed):
    B = x.shape[0] * x.shape[1]
    out = _make_gather(B)(x.reshape(B), embed)
    return out.reshape(x.shape[0], x.shape[1], _D)
